# baseline (speedup 1.0000x reference)
import functools

import jax
import jax.numpy as jnp
from jax import lax
from jax.experimental import pallas as pl
from jax.experimental.pallas import tpu as pltpu
from jax.experimental.pallas import tpu_sc as plsc

N = 10000
E = 160000
F_IN = 37
H = 12
C = 37
ED = 20
HID = F_IN * H
FEATS = 1500
ODIM = 128
G = 512

NC = 2
NS = 16
NW = NC * NS
KD = 1024
E2 = E + N
E2P = 196608
JPT = E2P // (NW * KD)
NP = 10240
NPT = NP // NS

CP = 48
GC = 6
CW = 80


def _gat_msg_body(src2_ref, dst2_ref, xp_ref, exT_ref, denT_ref,
                  h1p_ref, alphaT_ref,
                  idx_s, idx_d, idxp, rows, exb, alb, den_vm, zbuf, accum, sem):
    c = lax.axis_index("c")
    s = lax.axis_index("s")
    wid = s * NC + c

    def zb(i, _):
        for w in range(CP // 16):
            zbuf[i, pl.ds(w * 16, 16)] = jnp.zeros((16,), jnp.float32)
        return _
    lax.fori_loop(0, NPT, zb, None)

    def head_pass(h, _):
        pltpu.sync_copy(zbuf, accum.at[pl.ds(pl.multiple_of(s * NPT, 128), NPT)])
        pltpu.sync_copy(denT_ref.at[pl.ds(pl.multiple_of(h * NP, 1024), NP)], den_vm)
        plsc.subcore_barrier()

        def chunk_loop(j, _):
            ebase = pl.multiple_of(wid * (JPT * KD) + j * KD, 1024)
            rbase = pl.multiple_of(ebase // 128, 8)
            pltpu.sync_copy(src2_ref.at[pl.ds(rbase, 8)], idx_s)
            pltpu.sync_copy(dst2_ref.at[pl.ds(rbase, 8)], idx_d)
            pltpu.sync_copy(exT_ref.at[pl.ds(pl.multiple_of(h * E2P + ebase, 1024), KD)], exb)

            hN = h * NP

            def vec_body(i, _):
                g = i // 8
                l = i % 8
                sl = pl.ds(l * 16, 16)
                se = pl.ds(i * 16, 16)
                idxp[g, sl] = idx_s[g, sl] + hN
                dv = plsc.load_gather(den_vm, [idx_d[g, sl]])
                alb[se] = exb[se] / (dv + 1e-16)
                return _
            lax.fori_loop(0, KD // 16, vec_body, None)

            pltpu.sync_copy(alb, alphaT_ref.at[pl.ds(pl.multiple_of(h * E2P + ebase, 1024), KD)])

            descs = [pltpu.async_copy(xp_ref.at[idxp.at[g]],
                                      rows.at[pl.ds(g * 128, 128)], sem)
                     for g in range(8)]
            for d in descs:
                d.wait()

            @plsc.parallel_loop(0, KD, unroll=8)
            def _mul(k):
                a = plsc.load_gather(alb, [jnp.full((16,), k, jnp.int32)])
                for w in range(CP // 16):
                    sl = pl.ds(w * 16, 16)
                    rows[k, sl] = rows[k, sl] * a

            for g in range(8):
                pltpu.sync_copy(rows.at[pl.ds(g * 128, 128)],
                                accum.at[idx_d.at[g]], add=True)
            return _
        lax.fori_loop(0, JPT, chunk_loop, None)

        plsc.subcore_barrier()
        obase = pl.multiple_of((c * H + h) * NP + s * NPT, 128)
        pltpu.sync_copy(accum.at[pl.ds(pl.multiple_of(s * NPT, 128), NPT)],
                        h1p_ref.at[pl.ds(obase, NPT)])
        return _
    lax.fori_loop(0, H, head_pass, None)


@jax.jit
def _gat_msg(src2g, dst2g, xp_flat, exT, denT):
    return pl.kernel(
        _gat_msg_body,
        out_type=(
            jax.ShapeDtypeStruct((NC * H * NP, CP), jnp.float32),
            jax.ShapeDtypeStruct((H * E2P,), jnp.float32),
        ),
        mesh=plsc.VectorSubcoreMesh(core_axis_name="c", subcore_axis_name="s"),
        compiler_params=pltpu.CompilerParams(
            needs_layout_passes=False, use_tc_tiling_on_sc=False),
        scratch_types=[
            pltpu.VMEM((8, 128), jnp.int32),
            pltpu.VMEM((8, 128), jnp.int32),
            pltpu.VMEM((8, 128), jnp.int32),
            pltpu.VMEM((KD, CP), jnp.float32),
            pltpu.VMEM((KD,), jnp.float32),
            pltpu.VMEM((KD,), jnp.float32),
            pltpu.VMEM((NP,), jnp.float32),
            pltpu.VMEM((NPT, CP), jnp.float32),
            pltpu.VMEM_SHARED((NP, CP), jnp.float32),
            pltpu.SemaphoreType.DMA,
        ],
    )(src2g, dst2g, xp_flat, exT, denT)


def _edge_soft_body(src2_ref, dst2_ref, asrc_ref, adst_ref, ae_ref, ew_ref,
                    exT_ref, sd_ref,
                    idx_s, idx_d, sbuf, dbuf, aeb, ewb, buf16, exT_buf,
                    accum, sem):
    c = lax.axis_index("c")
    s = lax.axis_index("s")
    wid = s * NC + c

    def zb(k, _):
        buf16[k, pl.ds(0, 16)] = jnp.zeros((16,), jnp.float32)
        return _
    lax.fori_loop(0, KD, zb, None)
    pltpu.sync_copy(buf16.at[pl.ds(0, NPT)],
                    accum.at[pl.ds(pl.multiple_of(s * NPT, 128), NPT)])
    plsc.subcore_barrier()

    io16 = lax.iota(jnp.int32, 16)

    def chunk_loop(j, _):
        ebase = pl.multiple_of(wid * (JPT * KD) + j * KD, 1024)
        rbase = pl.multiple_of(ebase // 128, 8)
        pltpu.sync_copy(src2_ref.at[pl.ds(rbase, 8)], idx_s)
        pltpu.sync_copy(dst2_ref.at[pl.ds(rbase, 8)], idx_d)
        pltpu.sync_copy(ae_ref.at[pl.ds(ebase, KD)], aeb)
        pltpu.sync_copy(ew_ref.at[pl.ds(ebase, KD)], ewb)
        descs = [pltpu.async_copy(asrc_ref.at[idx_s.at[g]],
                                  sbuf.at[pl.ds(g * 128, 128)], sem)
                 for g in range(8)]
        descs += [pltpu.async_copy(adst_ref.at[idx_d.at[g]],
                                   dbuf.at[pl.ds(g * 128, 128)], sem)
                  for g in range(8)]
        for d in descs:
            d.wait()

        @plsc.parallel_loop(0, KD, unroll=8)
        def _soft(k):
            sl = pl.ds(0, 16)
            zrow = sbuf[k, sl] + dbuf[k, sl] + aeb[k, sl]
            zrow = jnp.where(zrow >= 0.0, zrow, 0.2 * zrow)
            exr = jnp.exp(zrow)
            ewk = plsc.load_gather(ewb, [jnp.full((16,), k, jnp.int32)])
            row = jnp.where(io16 < 12, exr,
                            jnp.where(io16 == 12, ewk,
                                      jnp.zeros((16,), jnp.float32)))
            vmask = jnp.where(
                jnp.full((16,), ebase + k - E2, jnp.int32) < 0, 1.0, 0.0)
            buf16[k, sl] = row * vmask

        @plsc.parallel_loop(0, H * (KD // 16), unroll=8)
        def _tw(i):
            h = i // (KD // 16)
            kw = i % (KD // 16)
            v = plsc.load_gather(
                buf16, [io16 + kw * 16, jnp.full((16,), h, jnp.int32)])
            exT_buf[h, pl.ds(kw * 16, 16)] = v
        def hw_body(h, _):
            pltpu.sync_copy(
                exT_buf.at[h],
                exT_ref.at[pl.ds(pl.multiple_of(h * E2P + ebase, 1024), KD)])
            return _
        lax.fori_loop(0, H, hw_body, None)

        for g in range(8):
            pltpu.sync_copy(buf16.at[pl.ds(g * 128, 128)],
                            accum.at[idx_d.at[g]], add=True)
        return _
    lax.fori_loop(0, JPT, chunk_loop, None)

    plsc.subcore_barrier()
    obase = pl.multiple_of(c * NP + s * NPT, 128)
    pltpu.sync_copy(accum.at[pl.ds(pl.multiple_of(s * NPT, 128), NPT)],
                    sd_ref.at[pl.ds(obase, NPT)])


@jax.jit
def _edge_soft(src2g, dst2g, asrcp, adstp, aep, ewp):
    return pl.kernel(
        _edge_soft_body,
        out_type=(
            jax.ShapeDtypeStruct((H * E2P,), jnp.float32),
            jax.ShapeDtypeStruct((NC * NP, 16), jnp.float32),
        ),
        mesh=plsc.VectorSubcoreMesh(core_axis_name="c", subcore_axis_name="s"),
        compiler_params=pltpu.CompilerParams(
            needs_layout_passes=False, use_tc_tiling_on_sc=False),
        scratch_types=[
            pltpu.VMEM((8, 128), jnp.int32),
            pltpu.VMEM((8, 128), jnp.int32),
            pltpu.VMEM((KD, 16), jnp.float32),
            pltpu.VMEM((KD, 16), jnp.float32),
            pltpu.VMEM((KD, 16), jnp.float32),
            pltpu.VMEM((KD,), jnp.float32),
            pltpu.VMEM((KD, 16), jnp.float32),
            pltpu.VMEM((H, KD), jnp.float32),
            pltpu.VMEM_SHARED((NP, 16), jnp.float32),
            pltpu.SemaphoreType.DMA,
        ],
    )(src2g, dst2g, asrcp, adstp, aep, ewp)


def _gcn_msg_body(src2_ref, dst2_ref, xw_ref, ew_ref, dinv_ref,
                  h2p_ref,
                  idx_s, idx_d, rows, ewb, nrm, dinv_vm, accum, sem):
    c = lax.axis_index("c")
    s = lax.axis_index("s")
    wid = s * NC + c

    pltpu.sync_copy(dinv_ref, dinv_vm)

    def chunk_pass(jc, _):
        def zb(k, _):
            for w in range(CW // 16):
                rows[k, pl.ds(w * 16, 16)] = jnp.zeros((16,), jnp.float32)
            return _
        lax.fori_loop(0, KD // 2, zb, None)
        pltpu.sync_copy(rows, accum.at[pl.ds(pl.multiple_of(s * NPT, 128), KD // 2)])
        pltpu.sync_copy(rows.at[pl.ds(0, NPT - KD // 2)],
                        accum.at[pl.ds(pl.multiple_of(s * NPT + KD // 2, 128), NPT - KD // 2)])
        plsc.subcore_barrier()

        jN = jc * NP

        def chunk_loop(j, _):
            ebase = pl.multiple_of(wid * (JPT * KD) + j * KD, 1024)
            rbase = pl.multiple_of(ebase // 128, 8)
            pltpu.sync_copy(src2_ref.at[pl.ds(rbase, 8)], idx_s)
            pltpu.sync_copy(dst2_ref.at[pl.ds(rbase, 8)], idx_d)
            pltpu.sync_copy(ew_ref.at[pl.ds(ebase, KD)], ewb)

            def vec_body(i, _):
                g = i // 8
                l = i % 8
                sl = pl.ds(l * 16, 16)
                se = pl.ds(i * 16, 16)
                sv = idx_s[g, sl]
                ds_ = plsc.load_gather(dinv_vm, [sv])
                dd_ = plsc.load_gather(dinv_vm, [idx_d[g, sl]])
                nrm[se] = ds_ * ewb[se] * dd_
                idx_s[g, sl] = sv + jN
                return _
            lax.fori_loop(0, KD // 16, vec_body, None)

            for q in range(2):
                descs = [pltpu.async_copy(xw_ref.at[idx_s.at[q * 4 + g]],
                                          rows.at[pl.ds(g * 128, 128)], sem)
                         for g in range(4)]
                for d in descs:
                    d.wait()

                @plsc.parallel_loop(0, KD // 2, unroll=8)
                def _mul(k):
                    a = plsc.load_gather(
                        nrm, [jnp.full((16,), k + q * (KD // 2), jnp.int32)])
                    for w in range(CW // 16):
                        sl = pl.ds(w * 16, 16)
                        rows[k, sl] = rows[k, sl] * a

                for g in range(4):
                    pltpu.sync_copy(rows.at[pl.ds(g * 128, 128)],
                                    accum.at[idx_d.at[q * 4 + g]], add=True)
            return _
        lax.fori_loop(0, JPT, chunk_loop, None)

        plsc.subcore_barrier()
        obase = pl.multiple_of((c * GC + jc) * NP + s * NPT, 128)
        pltpu.sync_copy(accum.at[pl.ds(pl.multiple_of(s * NPT, 128), NPT)],
                        h2p_ref.at[pl.ds(obase, NPT)])
        plsc.subcore_barrier()
        return _
    lax.fori_loop(0, GC, chunk_pass, None)


@jax.jit
def _gcn_msg(src2g, dst2g, xw_flat, ewp, dinv):
    return pl.kernel(
        _gcn_msg_body,
        out_type=jax.ShapeDtypeStruct((NC * GC * NP, CW), jnp.float32),
        mesh=plsc.VectorSubcoreMesh(core_axis_name="c", subcore_axis_name="s"),
        compiler_params=pltpu.CompilerParams(
            needs_layout_passes=False, use_tc_tiling_on_sc=False),
        scratch_types=[
            pltpu.VMEM((8, 128), jnp.int32),
            pltpu.VMEM((8, 128), jnp.int32),
            pltpu.VMEM((KD // 2, CW), jnp.float32),
            pltpu.VMEM((KD,), jnp.float32),
            pltpu.VMEM((KD,), jnp.float32),
            pltpu.VMEM((NP,), jnp.float32),
            pltpu.VMEM_SHARED((NP, CW), jnp.float32),
            pltpu.SemaphoreType.DMA,
        ],
    )(src2g, dst2g, xw_flat, ewp, dinv)


def _mlp_head_kernel(h_ref, wg1_ref, bg1_ref, wg2_ref, bg2_ref,
                     wf1_ref, bf1_ref, wo_ref, bo_ref, out_ref):
    h = h_ref[...]
    t = jnp.maximum(h @ wg1_ref[...] + bg1_ref[...][None, :], 0.0)
    t = t @ wg2_ref[...] + bg2_ref[...][None, :]
    t = jnp.maximum(t @ wf1_ref[...] + bf1_ref[...][None, :], 0.0)
    o = t @ wo_ref[...] + bo_ref[...][None, :]
    out_ref[...] = jax.nn.sigmoid(o)


def _mlp_head(h, Wg1, bg1, Wg2, bg2, Wf1, bf1, Wo, bo):
    return pl.pallas_call(
        _mlp_head_kernel,
        out_shape=jax.ShapeDtypeStruct((G, 1), jnp.float32),
    )(h, Wg1, bg1, Wg2, bg2, Wf1, bf1, Wo, bo)


def kernel(x, edge_index, batch, edge_attr, edge_weight, W1, att_src, att_dst,
           We, att_edge, b1, W2, b2, Wg1, bg1, Wg2, bg2, Wf1, bf1, Wo, bo):
    src, dst = edge_index[0], edge_index[1]
    loop = jnp.arange(N, dtype=edge_index.dtype)
    src2 = jnp.concatenate([src, loop])
    dst2 = jnp.concatenate([dst, loop])
    src2g = jnp.pad(src2, (0, E2P - E2)).reshape(E2P // 128, 128)
    dst2g = jnp.pad(dst2, (0, E2P - E2)).reshape(E2P // 128, 128)

    cnt = jax.ops.segment_sum(jnp.ones((E,), jnp.float32), dst, num_segments=N)
    ea_mean = jax.ops.segment_sum(edge_attr, dst, num_segments=N) / jnp.maximum(cnt, 1.0)[:, None]

    Ms = (W1.reshape(F_IN, H, C) * att_src[0][None]).sum(-1)
    Md = (W1.reshape(F_IN, H, C) * att_dst[0][None]).sum(-1)
    Me = (We.reshape(ED, H, C) * att_edge[0][None]).sum(-1)
    xp = (x @ W1).reshape(N, H, C)
    a_src = x @ Ms
    a_dst = x @ Md
    a_edge = jnp.concatenate([edge_attr @ Me, ea_mean @ Me], axis=0)

    ew2 = jnp.concatenate([edge_weight, jnp.ones((N,), jnp.float32)])
    ewp = jnp.pad(ew2, (0, E2P - E2))
    asrcp = jnp.pad(a_src, ((0, NP - N), (0, 4)))
    adstp = jnp.pad(a_dst, ((0, NP - N), (0, 4)))
    aep = jnp.pad(a_edge, ((0, E2P - E2), (0, 4)))
    exT, sd_part = _edge_soft(src2g, dst2g, asrcp, adstp, aep, ewp)
    sd_tot = sd_part.reshape(NC, NP, 16).sum(0)
    denT = sd_tot[:, :12].T.reshape(H * NP)
    deg = sd_tot[:N, 12]

    xp_flat = jnp.pad(xp.transpose(1, 0, 2),
                      ((0, 0), (0, NP - N), (0, CP - C))).reshape(H * NP, CP)

    h1p, alphaT = _gat_msg(src2g, dst2g, xp_flat, exT, denT)
    alpha = alphaT.reshape(H, E2P)[:, :E2].T
    h1 = h1p.reshape(NC, H, NP, CP).sum(0)[:, :N, :C]
    h1 = h1.transpose(1, 0, 2).reshape(N, HID) + b1
    x1 = jnp.maximum(h1, 0.0)

    dinv = jnp.where(deg > 0, deg ** -0.5, 0.0)
    dinvp = jnp.pad(dinv, (0, NP - N))
    xw = x1 @ W2
    xw_flat = jnp.pad(xw.reshape(N, GC, HID // GC).transpose(1, 0, 2),
                      ((0, 0), (0, NP - N), (0, CW - HID // GC))).reshape(GC * NP, CW)
    h2p = _gcn_msg(src2g, dst2g, xw_flat, ewp, dinvp)
    h2 = h2p.reshape(NC, GC, NP, CW).sum(0)[:, :N, :HID // GC]
    h2 = h2.transpose(1, 0, 2).reshape(N, HID) + b2
    x2 = jnp.maximum(h2, 0.0)

    gmax = jax.ops.segment_max(x2, batch, num_segments=G)
    gmax = jnp.where(jnp.isfinite(gmax), gmax, 0.0)
    cg = jax.ops.segment_sum(jnp.ones((N,), jnp.float32), batch, num_segments=G)
    gmean = jax.ops.segment_sum(x2, batch, num_segments=G) / jnp.maximum(cg, 1.0)[:, None]
    h = jnp.concatenate([gmax, gmean], axis=1)

    out = _mlp_head(h, Wg1, bg1, Wg2, bg2, Wf1, bf1, Wo, bo)
    return out, alpha


# speedup vs baseline: 2.9296x; 2.9296x over previous
import functools

import jax
import jax.numpy as jnp
from jax import lax
from jax.experimental import pallas as pl
from jax.experimental.pallas import tpu as pltpu
from jax.experimental.pallas import tpu_sc as plsc

N = 10000
E = 160000
F_IN = 37
H = 12
C = 37
ED = 20
HID = F_IN * H
FEATS = 1500
ODIM = 128
G = 512

NC = 2
NS = 16
NW = NC * NS
KD = 1024
E2 = E + N
E2P = 196608
JPT = E2P // (NW * KD)
NP = 10240
NPT = NP // NS

CP = 48
GC = 6
CW = 80


def _gat_msg_body(src2_ref, dst2_ref, xp_ref, exT_ref, denT_ref,
                  h1p_ref, alphaT_ref,
                  idx_s, idx_d, idxp, rows, exb, alb, den_vm, zbuf, accum, sem):
    c = lax.axis_index("c")
    s = lax.axis_index("s")
    wid = s * NC + c

    def zb(i, _):
        for w in range(CP // 16):
            zbuf[i, pl.ds(w * 16, 16)] = jnp.zeros((16,), jnp.float32)
        return _
    lax.fori_loop(0, NPT, zb, None)

    def head_pass(h, _):
        pltpu.sync_copy(zbuf, accum.at[pl.ds(pl.multiple_of(s * NPT, 128), NPT)])
        pltpu.sync_copy(denT_ref.at[pl.ds(pl.multiple_of(h * NP, 1024), NP)], den_vm)
        plsc.subcore_barrier()

        def chunk_loop(j, _):
            ebase = pl.multiple_of(wid * (JPT * KD) + j * KD, 1024)

            @pl.when(ebase < E2)
            def _chunk():
                _gat_chunk(ebase)
            return _

        def _gat_chunk(ebase):
            rbase = pl.multiple_of(ebase // 128, 8)
            pltpu.sync_copy(src2_ref.at[pl.ds(rbase, 8)], idx_s)
            pltpu.sync_copy(dst2_ref.at[pl.ds(rbase, 8)], idx_d)
            pltpu.sync_copy(exT_ref.at[pl.ds(pl.multiple_of(h * E2P + ebase, 1024), KD)], exb)

            hN = h * NP

            def vec_body(i, _):
                g = i // 8
                l = i % 8
                sl = pl.ds(l * 16, 16)
                se = pl.ds(i * 16, 16)
                idxp[g, sl] = idx_s[g, sl] + hN
                dv = plsc.load_gather(den_vm, [idx_d[g, sl]])
                alb[se] = exb[se] / (dv + 1e-16)
                return _
            lax.fori_loop(0, KD // 16, vec_body, None)

            pltpu.sync_copy(alb, alphaT_ref.at[pl.ds(pl.multiple_of(h * E2P + ebase, 1024), KD)])

            descs = [pltpu.async_copy(xp_ref.at[idxp.at[g]],
                                      rows.at[pl.ds(g * 128, 128)], sem)
                     for g in range(8)]
            for d in descs:
                d.wait()

            @plsc.parallel_loop(0, KD, unroll=8)
            def _mul(k):
                a = plsc.load_gather(alb, [jnp.full((16,), k, jnp.int32)])
                for w in range(CP // 16):
                    sl = pl.ds(w * 16, 16)
                    rows[k, sl] = rows[k, sl] * a

            for g in range(8):
                pltpu.sync_copy(rows.at[pl.ds(g * 128, 128)],
                                accum.at[idx_d.at[g]], add=True)
        lax.fori_loop(0, JPT, chunk_loop, None)

        plsc.subcore_barrier()
        obase = pl.multiple_of((c * H + h) * NP + s * NPT, 128)
        pltpu.sync_copy(accum.at[pl.ds(pl.multiple_of(s * NPT, 128), NPT)],
                        h1p_ref.at[pl.ds(obase, NPT)])
        return _
    lax.fori_loop(0, H, head_pass, None)


@jax.jit
def _gat_msg(src2g, dst2g, xp_flat, exT, denT):
    return pl.kernel(
        _gat_msg_body,
        out_type=(
            jax.ShapeDtypeStruct((NC * H * NP, CP), jnp.float32),
            jax.ShapeDtypeStruct((H * E2P,), jnp.float32),
        ),
        mesh=plsc.VectorSubcoreMesh(core_axis_name="c", subcore_axis_name="s"),
        compiler_params=pltpu.CompilerParams(
            needs_layout_passes=False, use_tc_tiling_on_sc=False),
        scratch_types=[
            pltpu.VMEM((8, 128), jnp.int32),
            pltpu.VMEM((8, 128), jnp.int32),
            pltpu.VMEM((8, 128), jnp.int32),
            pltpu.VMEM((KD, CP), jnp.float32),
            pltpu.VMEM((KD,), jnp.float32),
            pltpu.VMEM((KD,), jnp.float32),
            pltpu.VMEM((NP,), jnp.float32),
            pltpu.VMEM((NPT, CP), jnp.float32),
            pltpu.VMEM_SHARED((NP, CP), jnp.float32),
            pltpu.SemaphoreType.DMA,
        ],
    )(src2g, dst2g, xp_flat, exT, denT)


def _edge_soft_body(src2_ref, dst2_ref, asrc_ref, adst_ref, ae_ref, ew_ref,
                    exT_ref, sd_ref,
                    idx_s, idx_d, sbuf, dbuf, aeb, ewb, buf16, exT_buf,
                    accum, sem):
    c = lax.axis_index("c")
    s = lax.axis_index("s")
    wid = s * NC + c

    def zb(k, _):
        buf16[k, pl.ds(0, 16)] = jnp.zeros((16,), jnp.float32)
        return _
    lax.fori_loop(0, KD, zb, None)
    pltpu.sync_copy(buf16.at[pl.ds(0, NPT)],
                    accum.at[pl.ds(pl.multiple_of(s * NPT, 128), NPT)])
    plsc.subcore_barrier()

    io16 = lax.iota(jnp.int32, 16)

    def chunk_loop(j, _):
        ebase = pl.multiple_of(wid * (JPT * KD) + j * KD, 1024)

        @pl.when(ebase < E2)
        def _chunk():
            _soft_chunk(ebase)
        return _

    def _soft_chunk(ebase):
        rbase = pl.multiple_of(ebase // 128, 8)
        pltpu.sync_copy(src2_ref.at[pl.ds(rbase, 8)], idx_s)
        pltpu.sync_copy(dst2_ref.at[pl.ds(rbase, 8)], idx_d)
        pltpu.sync_copy(ae_ref.at[pl.ds(ebase, KD)], aeb)
        pltpu.sync_copy(ew_ref.at[pl.ds(ebase, KD)], ewb)
        descs = [pltpu.async_copy(asrc_ref.at[idx_s.at[g]],
                                  sbuf.at[pl.ds(g * 128, 128)], sem)
                 for g in range(8)]
        descs += [pltpu.async_copy(adst_ref.at[idx_d.at[g]],
                                   dbuf.at[pl.ds(g * 128, 128)], sem)
                  for g in range(8)]
        for d in descs:
            d.wait()

        @plsc.parallel_loop(0, KD, unroll=8)
        def _soft(k):
            sl = pl.ds(0, 16)
            zrow = sbuf[k, sl] + dbuf[k, sl] + aeb[k, sl]
            zrow = jnp.where(zrow >= 0.0, zrow, 0.2 * zrow)
            exr = jnp.exp(zrow)
            ewk = plsc.load_gather(ewb, [jnp.full((16,), k, jnp.int32)])
            row = jnp.where(io16 < 12, exr,
                            jnp.where(io16 == 12, ewk,
                                      jnp.zeros((16,), jnp.float32)))
            vmask = jnp.where(
                jnp.full((16,), ebase + k - E2, jnp.int32) < 0, 1.0, 0.0)
            buf16[k, sl] = row * vmask

        @plsc.parallel_loop(0, H * (KD // 16), unroll=8)
        def _tw(i):
            h = i // (KD // 16)
            kw = i % (KD // 16)
            v = plsc.load_gather(
                buf16, [io16 + kw * 16, jnp.full((16,), h, jnp.int32)])
            exT_buf[h, pl.ds(kw * 16, 16)] = v
        def hw_body(h, _):
            pltpu.sync_copy(
                exT_buf.at[h],
                exT_ref.at[pl.ds(pl.multiple_of(h * E2P + ebase, 1024), KD)])
            return _
        lax.fori_loop(0, H, hw_body, None)

        for g in range(8):
            pltpu.sync_copy(buf16.at[pl.ds(g * 128, 128)],
                            accum.at[idx_d.at[g]], add=True)
    lax.fori_loop(0, JPT, chunk_loop, None)

    plsc.subcore_barrier()
    obase = pl.multiple_of(c * NP + s * NPT, 128)
    pltpu.sync_copy(accum.at[pl.ds(pl.multiple_of(s * NPT, 128), NPT)],
                    sd_ref.at[pl.ds(obase, NPT)])


@jax.jit
def _edge_soft(src2g, dst2g, asrcp, adstp, aep, ewp):
    return pl.kernel(
        _edge_soft_body,
        out_type=(
            jax.ShapeDtypeStruct((H * E2P,), jnp.float32),
            jax.ShapeDtypeStruct((NC * NP, 16), jnp.float32),
        ),
        mesh=plsc.VectorSubcoreMesh(core_axis_name="c", subcore_axis_name="s"),
        compiler_params=pltpu.CompilerParams(
            needs_layout_passes=False, use_tc_tiling_on_sc=False),
        scratch_types=[
            pltpu.VMEM((8, 128), jnp.int32),
            pltpu.VMEM((8, 128), jnp.int32),
            pltpu.VMEM((KD, 16), jnp.float32),
            pltpu.VMEM((KD, 16), jnp.float32),
            pltpu.VMEM((KD, 16), jnp.float32),
            pltpu.VMEM((KD,), jnp.float32),
            pltpu.VMEM((KD, 16), jnp.float32),
            pltpu.VMEM((H, KD), jnp.float32),
            pltpu.VMEM_SHARED((NP, 16), jnp.float32),
            pltpu.SemaphoreType.DMA,
        ],
    )(src2g, dst2g, asrcp, adstp, aep, ewp)


def _gcn_msg_body(src2_ref, dst2_ref, xw_ref, ew_ref, dinv_ref,
                  h2p_ref,
                  idx_s, idx_d, rows, ewb, nrm, dinv_vm, accum, sem):
    c = lax.axis_index("c")
    s = lax.axis_index("s")
    wid = s * NC + c

    pltpu.sync_copy(dinv_ref, dinv_vm)

    def chunk_pass(jc, _):
        def zb(k, _):
            for w in range(CW // 16):
                rows[k, pl.ds(w * 16, 16)] = jnp.zeros((16,), jnp.float32)
            return _
        lax.fori_loop(0, KD // 2, zb, None)
        pltpu.sync_copy(rows, accum.at[pl.ds(pl.multiple_of(s * NPT, 128), KD // 2)])
        pltpu.sync_copy(rows.at[pl.ds(0, NPT - KD // 2)],
                        accum.at[pl.ds(pl.multiple_of(s * NPT + KD // 2, 128), NPT - KD // 2)])
        plsc.subcore_barrier()

        jN = jc * NP

        def chunk_loop(j, _):
            ebase = pl.multiple_of(wid * (JPT * KD) + j * KD, 1024)

            @pl.when(ebase < E2)
            def _chunk():
                _gcn_chunk(ebase)
            return _

        def _gcn_chunk(ebase):
            rbase = pl.multiple_of(ebase // 128, 8)
            pltpu.sync_copy(src2_ref.at[pl.ds(rbase, 8)], idx_s)
            pltpu.sync_copy(dst2_ref.at[pl.ds(rbase, 8)], idx_d)
            pltpu.sync_copy(ew_ref.at[pl.ds(ebase, KD)], ewb)

            def vec_body(i, _):
                g = i // 8
                l = i % 8
                sl = pl.ds(l * 16, 16)
                se = pl.ds(i * 16, 16)
                sv = idx_s[g, sl]
                ds_ = plsc.load_gather(dinv_vm, [sv])
                dd_ = plsc.load_gather(dinv_vm, [idx_d[g, sl]])
                nrm[se] = ds_ * ewb[se] * dd_
                idx_s[g, sl] = sv + jN
                return _
            lax.fori_loop(0, KD // 16, vec_body, None)

            for q in range(2):
                descs = [pltpu.async_copy(xw_ref.at[idx_s.at[q * 4 + g]],
                                          rows.at[pl.ds(g * 128, 128)], sem)
                         for g in range(4)]
                for d in descs:
                    d.wait()

                @plsc.parallel_loop(0, KD // 2, unroll=8)
                def _mul(k):
                    a = plsc.load_gather(
                        nrm, [jnp.full((16,), k + q * (KD // 2), jnp.int32)])
                    for w in range(CW // 16):
                        sl = pl.ds(w * 16, 16)
                        rows[k, sl] = rows[k, sl] * a

                for g in range(4):
                    pltpu.sync_copy(rows.at[pl.ds(g * 128, 128)],
                                    accum.at[idx_d.at[q * 4 + g]], add=True)
        lax.fori_loop(0, JPT, chunk_loop, None)

        plsc.subcore_barrier()
        obase = pl.multiple_of((c * GC + jc) * NP + s * NPT, 128)
        pltpu.sync_copy(accum.at[pl.ds(pl.multiple_of(s * NPT, 128), NPT)],
                        h2p_ref.at[pl.ds(obase, NPT)])
        plsc.subcore_barrier()
        return _
    lax.fori_loop(0, GC, chunk_pass, None)


@jax.jit
def _gcn_msg(src2g, dst2g, xw_flat, ewp, dinv):
    return pl.kernel(
        _gcn_msg_body,
        out_type=jax.ShapeDtypeStruct((NC * GC * NP, CW), jnp.float32),
        mesh=plsc.VectorSubcoreMesh(core_axis_name="c", subcore_axis_name="s"),
        compiler_params=pltpu.CompilerParams(
            needs_layout_passes=False, use_tc_tiling_on_sc=False),
        scratch_types=[
            pltpu.VMEM((8, 128), jnp.int32),
            pltpu.VMEM((8, 128), jnp.int32),
            pltpu.VMEM((KD // 2, CW), jnp.float32),
            pltpu.VMEM((KD,), jnp.float32),
            pltpu.VMEM((KD,), jnp.float32),
            pltpu.VMEM((NP,), jnp.float32),
            pltpu.VMEM_SHARED((NP, CW), jnp.float32),
            pltpu.SemaphoreType.DMA,
        ],
    )(src2g, dst2g, xw_flat, ewp, dinv)


def _mlp_head_kernel(h_ref, wg1_ref, bg1_ref, wg2_ref, bg2_ref,
                     wf1_ref, bf1_ref, wo_ref, bo_ref, out_ref):
    h = h_ref[...]
    t = jnp.maximum(h @ wg1_ref[...] + bg1_ref[...][None, :], 0.0)
    t = t @ wg2_ref[...] + bg2_ref[...][None, :]
    t = jnp.maximum(t @ wf1_ref[...] + bf1_ref[...][None, :], 0.0)
    o = t @ wo_ref[...] + bo_ref[...][None, :]
    out_ref[...] = jax.nn.sigmoid(o)


def _mlp_head(h, Wg1, bg1, Wg2, bg2, Wf1, bf1, Wo, bo):
    return pl.pallas_call(
        _mlp_head_kernel,
        out_shape=jax.ShapeDtypeStruct((G, 1), jnp.float32),
    )(h, Wg1, bg1, Wg2, bg2, Wf1, bf1, Wo, bo)


def kernel(x, edge_index, batch, edge_attr, edge_weight, W1, att_src, att_dst,
           We, att_edge, b1, W2, b2, Wg1, bg1, Wg2, bg2, Wf1, bf1, Wo, bo):
    src, dst = edge_index[0], edge_index[1]
    loop = jnp.arange(N, dtype=edge_index.dtype)
    src2 = jnp.concatenate([src, loop])
    dst2 = jnp.concatenate([dst, loop])
    src2g = jnp.pad(src2, (0, E2P - E2)).reshape(E2P // 128, 128)
    dst2g = jnp.pad(dst2, (0, E2P - E2)).reshape(E2P // 128, 128)

    cnt = jax.ops.segment_sum(jnp.ones((E,), jnp.float32), dst, num_segments=N)
    ea_mean = jax.ops.segment_sum(edge_attr, dst, num_segments=N) / jnp.maximum(cnt, 1.0)[:, None]

    Ms = (W1.reshape(F_IN, H, C) * att_src[0][None]).sum(-1)
    Md = (W1.reshape(F_IN, H, C) * att_dst[0][None]).sum(-1)
    Me = (We.reshape(ED, H, C) * att_edge[0][None]).sum(-1)
    xp = (x @ W1).reshape(N, H, C)
    a_src = x @ Ms
    a_dst = x @ Md
    a_edge = jnp.concatenate([edge_attr @ Me, ea_mean @ Me], axis=0)

    ew2 = jnp.concatenate([edge_weight, jnp.ones((N,), jnp.float32)])
    ewp = jnp.pad(ew2, (0, E2P - E2))
    asrcp = jnp.pad(a_src, ((0, NP - N), (0, 4)))
    adstp = jnp.pad(a_dst, ((0, NP - N), (0, 4)))
    aep = jnp.pad(a_edge, ((0, E2P - E2), (0, 4)))
    exT, sd_part = _edge_soft(src2g, dst2g, asrcp, adstp, aep, ewp)
    sd_tot = sd_part.reshape(NC, NP, 16).sum(0)
    denT = sd_tot[:, :12].T.reshape(H * NP)
    deg = sd_tot[:N, 12]

    xp_flat = jnp.pad(xp.transpose(1, 0, 2),
                      ((0, 0), (0, NP - N), (0, CP - C))).reshape(H * NP, CP)

    h1p, alphaT = _gat_msg(src2g, dst2g, xp_flat, exT, denT)
    alpha = alphaT.reshape(H, E2P)[:, :E2].T
    h1 = h1p.reshape(NC, H, NP, CP).sum(0)[:, :N, :C]
    h1 = h1.transpose(1, 0, 2).reshape(N, HID) + b1
    x1 = jnp.maximum(h1, 0.0)

    dinv = jnp.where(deg > 0, deg ** -0.5, 0.0)
    dinvp = jnp.pad(dinv, (0, NP - N))
    xw = x1 @ W2
    xw_flat = jnp.pad(xw.reshape(N, GC, HID // GC).transpose(1, 0, 2),
                      ((0, 0), (0, NP - N), (0, CW - HID // GC))).reshape(GC * NP, CW)
    h2p = _gcn_msg(src2g, dst2g, xw_flat, ewp, dinvp)
    h2 = h2p.reshape(NC, GC, NP, CW).sum(0)[:, :N, :HID // GC]
    h2 = h2.transpose(1, 0, 2).reshape(N, HID) + b2
    x2 = jnp.maximum(h2, 0.0)

    gmax = jax.ops.segment_max(x2, batch, num_segments=G)
    gmax = jnp.where(jnp.isfinite(gmax), gmax, 0.0)
    cg = jax.ops.segment_sum(jnp.ones((N,), jnp.float32), batch, num_segments=G)
    gmean = jax.ops.segment_sum(x2, batch, num_segments=G) / jnp.maximum(cg, 1.0)[:, None]
    h = jnp.concatenate([gmax, gmean], axis=1)

    out = _mlp_head(h, Wg1, bg1, Wg2, bg2, Wf1, bf1, Wo, bo)
    return out, alpha
